# Initial kernel scaffold; baseline (speedup 1.0000x reference)
import jax
import jax.numpy as jnp
from jax.experimental import pallas as pl


def kernel(U, neuron_regions, eids, r_map, neuron_slot, region_emb, eid_emb, ln_g, ln_b, W1, b1, W2, b2):
    raise NotImplementedError("write your pallas kernel here")


# speedup vs baseline: 14768.7482x; 14768.7482x over previous
import functools
import math

import jax
import jax.numpy as jnp
from jax import lax
from jax.experimental import pallas as pl
from jax.experimental.pallas import tpu as pltpu


def _decoder_body(ut_ref, nr_ref, eids_ref, rmap_ref, nsT_ref, reT_ref,
                  eeT_ref, lng_ref, lnb_ref, w1t_ref, b1_ref, w2wt_ref,
                  b2w_ref, w2b_ref, b2b_ref, out_ref):
    f32 = jnp.float32
    B, R, T, Ds = ut_ref.shape
    N = nr_ref.shape[1]
    n_regions = reT_ref.shape[1]
    n_eids = eeT_ref.shape[1]

    iota_r = lax.broadcasted_iota(jnp.int32, (R, n_regions), 0)
    RMT = (rmap_ref[...] == iota_r).astype(f32)

    iota_reg = lax.broadcasted_iota(jnp.int32, (n_regions, N), 0)
    iota_eid = lax.broadcasted_iota(jnp.int32, (n_eids, 1), 0)
    inv_sqrt2 = 1.0 / math.sqrt(2.0)

    for b in range(B):
        nr_row = nr_ref[pl.ds(b, 1), :]
        ohT = (nr_row == iota_reg).astype(f32)
        eid_oh = (eids_ref[b] == iota_eid).astype(f32)

        eT = (nsT_ref[...]
              + jnp.dot(reT_ref[...], ohT, preferred_element_type=f32)
              + jnp.dot(eeT_ref[...], eid_oh, preferred_element_type=f32))

        mu = jnp.mean(eT, axis=0, keepdims=True)
        xc = eT - mu
        var = jnp.mean(xc * xc, axis=0, keepdims=True)
        ehT = xc * lax.rsqrt(var + 1e-5) * lng_ref[...] + lnb_ref[...]

        pre = jnp.dot(w1t_ref[...], ehT, preferred_element_type=f32) + b1_ref[...]
        hT = 0.5 * pre * (1.0 + lax.erf(pre * inv_sqrt2))
        wT = jnp.dot(w2wt_ref[...], hT, preferred_element_type=f32) + b2w_ref[...]
        biasT = jnp.dot(w2b_ref[...], hT, preferred_element_type=f32) + b2b_ref[...]

        MT = jnp.dot(RMT, ohT, preferred_element_type=f32)

        u_flat = ut_ref[b].reshape(R * T, Ds)
        pall = jnp.dot(u_flat, wT, preferred_element_type=f32)
        acc = jnp.zeros((T, N), f32)
        for r in range(R):
            acc = acc + pall[r * T:(r + 1) * T, :] * MT[r:r + 1, :]
        out_ref[b] = acc + biasT


def kernel(U, neuron_regions, eids, r_map, neuron_slot, region_emb, eid_emb,
           ln_g, ln_b, W1, b1, W2, b2):
    B, T, R, Ds = U.shape
    N = neuron_regions.shape[1]
    d_id = neuron_slot.shape[1]

    ut = U.transpose(0, 2, 1, 3)
    nsT = neuron_slot[:N].T
    reT = region_emb.T
    eeT = eid_emb.T
    rmap_row = r_map.reshape(1, -1)
    lng = ln_g.reshape(-1, 1)
    lnb = ln_b.reshape(-1, 1)
    w1t = W1.T
    b1c = b1.reshape(-1, 1)
    w2wt = W2[:, :Ds].T
    b2w = b2[:Ds].reshape(-1, 1)
    w2b = W2[:, Ds].reshape(1, -1)
    b2b = b2[Ds:].reshape(1, 1)

    pred = pl.pallas_call(
        _decoder_body,
        out_shape=jax.ShapeDtypeStruct((B, T, N), jnp.float32),
        in_specs=[
            pl.BlockSpec(memory_space=pltpu.VMEM),
            pl.BlockSpec(memory_space=pltpu.VMEM),
            pl.BlockSpec(memory_space=pltpu.SMEM),
            pl.BlockSpec(memory_space=pltpu.VMEM),
            pl.BlockSpec(memory_space=pltpu.VMEM),
            pl.BlockSpec(memory_space=pltpu.VMEM),
            pl.BlockSpec(memory_space=pltpu.VMEM),
            pl.BlockSpec(memory_space=pltpu.VMEM),
            pl.BlockSpec(memory_space=pltpu.VMEM),
            pl.BlockSpec(memory_space=pltpu.VMEM),
            pl.BlockSpec(memory_space=pltpu.VMEM),
            pl.BlockSpec(memory_space=pltpu.VMEM),
            pl.BlockSpec(memory_space=pltpu.VMEM),
            pl.BlockSpec(memory_space=pltpu.VMEM),
            pl.BlockSpec(memory_space=pltpu.VMEM),
        ],
        out_specs=pl.BlockSpec(memory_space=pltpu.VMEM),
    )(ut, neuron_regions, eids, rmap_row, nsT, reT, eeT, lng, lnb,
      w1t, b1c, w2wt, b2w, w2b, b2b)
    return pred
